# baseline (device time: 124619 ns/iter reference)
import jax
import jax.numpy as jnp
from jax import lax
from jax.experimental import pallas as pl
from jax.experimental.pallas import tpu as pltpu

N_DEV = 4


def kernel(t, W):
    m_per, k = t.shape
    _, n = W.shape
    chunk = m_per // N_DEV

    def body(t_ref, w_ref, out_ref, send_buf, recv_buf,
             rs_send_sems, rs_recv_sems, ag_send_sems, ag_recv_sems):
        my = lax.axis_index("i")
        left = lax.rem(my + N_DEV - 1, N_DEV)
        right = lax.rem(my + 1, N_DEV)

        barrier_sem = pltpu.get_barrier_semaphore()
        for nbr in (left, right):
            pl.semaphore_signal(
                barrier_sem, inc=1,
                device_id=(nbr,), device_id_type=pl.DeviceIdType.MESH,
            )
        pl.semaphore_wait(barrier_sem, 2)

        def local_chunk_bf16(idx):
            return t_ref[pl.ds(idx * chunk, chunk), :].astype(jnp.bfloat16)

        send_buf[0, :, :] = local_chunk_bf16(my)
        s_chunk = None
        for h in range(N_DEV - 1):
            rdma = pltpu.make_async_remote_copy(
                src_ref=send_buf.at[h],
                dst_ref=recv_buf.at[h],
                send_sem=rs_send_sems.at[h],
                recv_sem=rs_recv_sems.at[h],
                device_id=(right,),
                device_id_type=pl.DeviceIdType.MESH,
            )
            rdma.start()
            rdma.wait()
            idx = lax.rem(my - (h + 1) + 2 * N_DEV, N_DEV)
            summed = recv_buf[h] + local_chunk_bf16(idx)
            if h < N_DEV - 2:
                send_buf[h + 1, :, :] = summed
            else:
                s_chunk = summed

        w_bf = w_ref[:, :].astype(jnp.bfloat16)
        out_chunk = jnp.dot(s_chunk, w_bf, preferred_element_type=jnp.float32)
        my_out = lax.rem(my + 1, N_DEV)
        out_ref[pl.ds(my_out * chunk, chunk), :] = out_chunk

        for h in range(N_DEV - 1):
            src_idx = lax.rem(my + 1 - h + 2 * N_DEV, N_DEV)
            rdma = pltpu.make_async_remote_copy(
                src_ref=out_ref.at[pl.ds(src_idx * chunk, chunk), :],
                dst_ref=out_ref.at[pl.ds(src_idx * chunk, chunk), :],
                send_sem=ag_send_sems.at[h],
                recv_sem=ag_recv_sems.at[h],
                device_id=(right,),
                device_id_type=pl.DeviceIdType.MESH,
            )
            rdma.start()
            rdma.wait()

    return pl.pallas_call(
        body,
        out_shape=jax.ShapeDtypeStruct((m_per, n), jnp.float32),
        in_specs=[
            pl.BlockSpec(memory_space=pltpu.VMEM),
            pl.BlockSpec(memory_space=pltpu.VMEM),
        ],
        out_specs=pl.BlockSpec(memory_space=pltpu.VMEM),
        scratch_shapes=[
            pltpu.VMEM((N_DEV - 1, chunk, k), jnp.bfloat16),
            pltpu.VMEM((N_DEV - 1, chunk, k), jnp.bfloat16),
            pltpu.SemaphoreType.DMA((N_DEV - 1,)),
            pltpu.SemaphoreType.DMA((N_DEV - 1,)),
            pltpu.SemaphoreType.DMA((N_DEV - 1,)),
            pltpu.SemaphoreType.DMA((N_DEV - 1,)),
        ],
        compiler_params=pltpu.CompilerParams(collective_id=0),
    )(t, W)


# device time: 57730 ns/iter; 2.1587x vs baseline; 2.1587x over previous
import jax
import jax.numpy as jnp
from jax import lax
from jax.experimental import pallas as pl
from jax.experimental.pallas import tpu as pltpu

N_DEV = 4


def kernel(t, W):
    m_per, k = t.shape
    _, n = W.shape
    chunk = m_per // N_DEV
    kh = k // 2
    nh = n // 2
    bf16 = jnp.bfloat16
    H = N_DEV - 1

    def body(t_ref, w_ref, out_ref,
             rs_send_r, rs_recv_r, rs_send_l, rs_recv_l,
             ag_own_r, ag_own_l, ag_recv_r, ag_recv_l,
             rs_r_ssem, rs_r_rsem, rs_l_ssem, rs_l_rsem,
             ag_r_ssem, ag_r_rsem, ag_l_ssem, ag_l_rsem):
        my = lax.axis_index("i")
        left = lax.rem(my + N_DEV - 1, N_DEV)
        right = lax.rem(my + 1, N_DEV)

        barrier_sem = pltpu.get_barrier_semaphore()
        for nbr in (left, right):
            pl.semaphore_signal(
                barrier_sem, inc=1,
                device_id=(nbr,), device_id_type=pl.DeviceIdType.MESH,
            )
        pl.semaphore_wait(barrier_sem, 2)

        def t_lo(idx):
            return t_ref[pl.ds(idx * chunk, chunk), :kh].astype(bf16)

        def t_hi(idx):
            return t_ref[pl.ds(idx * chunk, chunk), kh:].astype(bf16)

        def copy(src, dst, ssem, rsem, dev):
            return pltpu.make_async_remote_copy(
                src_ref=src, dst_ref=dst, send_sem=ssem, recv_sem=rsem,
                device_id=(dev,), device_id_type=pl.DeviceIdType.MESH,
            )

        rs_send_r[0, :, :] = t_lo(my)
        rs_send_l[0, :, :] = t_hi(lax.rem(my + 2, N_DEV))
        sum_lo = sum_hi = None
        for h in range(H):
            r = copy(rs_send_r.at[h], rs_recv_r.at[h],
                     rs_r_ssem.at[h], rs_r_rsem.at[h], right)
            l = copy(rs_send_l.at[h], rs_recv_l.at[h],
                     rs_l_ssem.at[h], rs_l_rsem.at[h], left)
            r.start()
            l.start()
            r.wait()
            l.wait()
            r_idx = lax.rem(my - h - 1 + 2 * N_DEV, N_DEV)
            l_idx = lax.rem(my + h + 3, N_DEV)
            summed_lo = rs_recv_r[h] + t_lo(r_idx)
            summed_hi = rs_recv_l[h] + t_hi(l_idx)
            if h < H - 1:
                rs_send_r[h + 1, :, :] = summed_lo
                rs_send_l[h + 1, :, :] = summed_hi
            else:
                sum_lo, sum_hi = summed_lo, summed_hi

        w_bf = w_ref[:, :].astype(bf16)
        out_chunk = (
            jnp.dot(sum_lo, w_bf[:kh, :], preferred_element_type=jnp.float32)
            + jnp.dot(sum_hi, w_bf[kh:, :], preferred_element_type=jnp.float32)
        )
        my_out = lax.rem(my + 1, N_DEV)
        out_ref[pl.ds(my_out * chunk, chunk), :] = out_chunk

        ag_own_r[:, :] = out_chunk[:, :nh].astype(bf16)
        ag_own_l[:, :] = out_chunk[:, nh:].astype(bf16)
        for h in range(H):
            src_r = ag_own_r if h == 0 else ag_recv_r.at[h - 1]
            src_l = ag_own_l if h == 0 else ag_recv_l.at[h - 1]
            r = copy(src_r, ag_recv_r.at[h],
                     ag_r_ssem.at[h], ag_r_rsem.at[h], right)
            l = copy(src_l, ag_recv_l.at[h],
                     ag_l_ssem.at[h], ag_l_rsem.at[h], left)
            r.start()
            l.start()
            r.wait()
            l.wait()
            r_idx = lax.rem(my - h + 2 * N_DEV, N_DEV)
            l_idx = lax.rem(my + 2 + h, N_DEV)
            out_ref[pl.ds(r_idx * chunk, chunk), :nh] = (
                ag_recv_r[h].astype(jnp.float32))
            out_ref[pl.ds(l_idx * chunk, chunk), nh:] = (
                ag_recv_l[h].astype(jnp.float32))

    return pl.pallas_call(
        body,
        out_shape=jax.ShapeDtypeStruct((m_per, n), jnp.float32),
        in_specs=[
            pl.BlockSpec(memory_space=pltpu.VMEM),
            pl.BlockSpec(memory_space=pltpu.VMEM),
        ],
        out_specs=pl.BlockSpec(memory_space=pltpu.VMEM),
        scratch_shapes=[
            pltpu.VMEM((H, chunk, kh), bf16),
            pltpu.VMEM((H, chunk, kh), bf16),
            pltpu.VMEM((H, chunk, kh), bf16),
            pltpu.VMEM((H, chunk, kh), bf16),
            pltpu.VMEM((chunk, nh), bf16),
            pltpu.VMEM((chunk, nh), bf16),
            pltpu.VMEM((H, chunk, nh), bf16),
            pltpu.VMEM((H, chunk, nh), bf16),
            pltpu.SemaphoreType.DMA((H,)),
            pltpu.SemaphoreType.DMA((H,)),
            pltpu.SemaphoreType.DMA((H,)),
            pltpu.SemaphoreType.DMA((H,)),
            pltpu.SemaphoreType.DMA((H,)),
            pltpu.SemaphoreType.DMA((H,)),
            pltpu.SemaphoreType.DMA((H,)),
            pltpu.SemaphoreType.DMA((H,)),
        ],
        compiler_params=pltpu.CompilerParams(collective_id=0),
    )(t, W)


# device time: 55748 ns/iter; 2.2354x vs baseline; 1.0356x over previous
import jax
import jax.numpy as jnp
from jax import lax
from jax.experimental import pallas as pl
from jax.experimental.pallas import tpu as pltpu

N_DEV = 4


def kernel(t, W):
    m_per, k = t.shape
    _, n = W.shape
    chunk = m_per // N_DEV
    kh = k // 2
    nh = n // 2
    bf16 = jnp.bfloat16
    H = N_DEV - 1

    def body(t_ref, w_ref, out_ref,
             rs_send_r, rs_recv_r, rs_send_l, rs_recv_l,
             rs_r_ssem, rs_r_rsem, rs_l_ssem, rs_l_rsem,
             ag_r_ssem, ag_r_rsem, ag_l_ssem, ag_l_rsem):
        my = lax.axis_index("i")
        left = lax.rem(my + N_DEV - 1, N_DEV)
        right = lax.rem(my + 1, N_DEV)

        barrier_sem = pltpu.get_barrier_semaphore()
        for nbr in (left, right):
            pl.semaphore_signal(
                barrier_sem, inc=1,
                device_id=(nbr,), device_id_type=pl.DeviceIdType.MESH,
            )
        pl.semaphore_wait(barrier_sem, 2)

        def t_lo(idx):
            return t_ref[pl.ds(idx * chunk, chunk), :kh].astype(bf16)

        def t_hi(idx):
            return t_ref[pl.ds(idx * chunk, chunk), kh:].astype(bf16)

        def copy(src, dst, ssem, rsem, dev):
            return pltpu.make_async_remote_copy(
                src_ref=src, dst_ref=dst, send_sem=ssem, recv_sem=rsem,
                device_id=(dev,), device_id_type=pl.DeviceIdType.MESH,
            )

        rs_send_r[0, :, :] = t_lo(my)
        rs_send_l[0, :, :] = t_hi(lax.rem(my + 2, N_DEV))
        sum_lo = sum_hi = None
        for h in range(H):
            r = copy(rs_send_r.at[h], rs_recv_r.at[h],
                     rs_r_ssem.at[h], rs_r_rsem.at[h], right)
            l = copy(rs_send_l.at[h], rs_recv_l.at[h],
                     rs_l_ssem.at[h], rs_l_rsem.at[h], left)
            r.start()
            l.start()
            r.wait()
            l.wait()
            r_idx = lax.rem(my - h - 1 + 2 * N_DEV, N_DEV)
            l_idx = lax.rem(my + h + 3, N_DEV)
            summed_lo = rs_recv_r[h] + t_lo(r_idx)
            summed_hi = rs_recv_l[h] + t_hi(l_idx)
            if h < H - 1:
                rs_send_r[h + 1, :, :] = summed_lo
                rs_send_l[h + 1, :, :] = summed_hi
            else:
                sum_lo, sum_hi = summed_lo, summed_hi

        w_bf = w_ref[:, :].astype(bf16)
        out_chunk = (
            jnp.dot(sum_lo, w_bf[:kh, :], preferred_element_type=jnp.float32)
            + jnp.dot(sum_hi, w_bf[kh:, :], preferred_element_type=jnp.float32)
        ).astype(bf16)
        my_out = lax.rem(my + 1, N_DEV)
        out_ref[pl.ds(my_out * chunk, chunk), :] = out_chunk

        for h in range(H):
            sr_idx = lax.rem(my + 1 - h + 2 * N_DEV, N_DEV)
            sl_idx = lax.rem(my + 1 + h, N_DEV)
            r_idx = lax.rem(my - h + 2 * N_DEV, N_DEV)
            l_idx = lax.rem(my + 2 + h, N_DEV)
            r = copy(out_ref.at[pl.ds(sr_idx * chunk, chunk), pl.ds(0, nh)],
                     out_ref.at[pl.ds(sr_idx * chunk, chunk), pl.ds(0, nh)],
                     ag_r_ssem.at[h], ag_r_rsem.at[h], right)
            l = copy(out_ref.at[pl.ds(sl_idx * chunk, chunk), pl.ds(nh, nh)],
                     out_ref.at[pl.ds(sl_idx * chunk, chunk), pl.ds(nh, nh)],
                     ag_l_ssem.at[h], ag_l_rsem.at[h], left)
            r.start()
            l.start()
            r.wait()
            l.wait()

    return pl.pallas_call(
        body,
        out_shape=jax.ShapeDtypeStruct((m_per, n), bf16),
        in_specs=[
            pl.BlockSpec(memory_space=pltpu.VMEM),
            pl.BlockSpec(memory_space=pltpu.VMEM),
        ],
        out_specs=pl.BlockSpec(memory_space=pltpu.VMEM),
        scratch_shapes=[
            pltpu.VMEM((H, chunk, kh), bf16),
            pltpu.VMEM((H, chunk, kh), bf16),
            pltpu.VMEM((H, chunk, kh), bf16),
            pltpu.VMEM((H, chunk, kh), bf16),
            pltpu.SemaphoreType.DMA((H,)),
            pltpu.SemaphoreType.DMA((H,)),
            pltpu.SemaphoreType.DMA((H,)),
            pltpu.SemaphoreType.DMA((H,)),
            pltpu.SemaphoreType.DMA((H,)),
            pltpu.SemaphoreType.DMA((H,)),
            pltpu.SemaphoreType.DMA((H,)),
            pltpu.SemaphoreType.DMA((H,)),
        ],
        compiler_params=pltpu.CompilerParams(collective_id=0),
    )(t, W)


# device time: 46569 ns/iter; 2.6760x vs baseline; 1.1971x over previous
import jax
import jax.numpy as jnp
from jax import lax
from jax.experimental import pallas as pl
from jax.experimental.pallas import tpu as pltpu

N_DEV = 4
S = 2


def kernel(t, W):
    m_per, k = t.shape
    _, n = W.shape
    chunk = m_per // N_DEV
    sh = chunk // S
    kh = k // 2
    nh = n // 2
    bf16 = jnp.bfloat16
    H = N_DEV - 1

    def body(t_ref, w_ref, out_ref,
             rs_send_r, rs_recv_r, rs_send_l, rs_recv_l,
             rs_r_ssem, rs_r_rsem, rs_l_ssem, rs_l_rsem,
             ag_r_ssem, ag_r_rsem, ag_l_ssem, ag_l_rsem):
        my = lax.axis_index("i")
        left = lax.rem(my + N_DEV - 1, N_DEV)
        right = lax.rem(my + 1, N_DEV)

        barrier_sem = pltpu.get_barrier_semaphore()
        for nbr in (left, right):
            pl.semaphore_signal(
                barrier_sem, inc=1,
                device_id=(nbr,), device_id_type=pl.DeviceIdType.MESH,
            )
        pl.semaphore_wait(barrier_sem, 2)

        def t_lo(idx, s):
            return t_ref[pl.ds(idx * chunk + s * sh, sh), :kh].astype(bf16)

        def t_hi(idx, s):
            return t_ref[pl.ds(idx * chunk + s * sh, sh), kh:].astype(bf16)

        def copy(src, dst, ssem, rsem, dev):
            return pltpu.make_async_remote_copy(
                src_ref=src, dst_ref=dst, send_sem=ssem, recv_sem=rsem,
                device_id=(dev,), device_id_type=pl.DeviceIdType.MESH,
            )

        def start_rs(s, h):
            r = copy(rs_send_r.at[s, h], rs_recv_r.at[s, h],
                     rs_r_ssem.at[s, h], rs_r_rsem.at[s, h], right)
            l = copy(rs_send_l.at[s, h], rs_recv_l.at[s, h],
                     rs_l_ssem.at[s, h], rs_l_rsem.at[s, h], left)
            r.start()
            l.start()
            return r, l

        def out_rows(idx, s):
            return pl.ds(idx * chunk + s * sh, sh)

        def start_ag(s, h):
            sr_idx = lax.rem(my + 1 - h + 2 * N_DEV, N_DEV)
            sl_idx = lax.rem(my + 1 + h, N_DEV)
            r = copy(out_ref.at[out_rows(sr_idx, s), pl.ds(0, nh)],
                     out_ref.at[out_rows(sr_idx, s), pl.ds(0, nh)],
                     ag_r_ssem.at[s, h], ag_r_rsem.at[s, h], right)
            l = copy(out_ref.at[out_rows(sl_idx, s), pl.ds(nh, nh)],
                     out_ref.at[out_rows(sl_idx, s), pl.ds(nh, nh)],
                     ag_l_ssem.at[s, h], ag_l_rsem.at[s, h], left)
            r.start()
            l.start()
            return r, l

        rs = {}
        ag = {}

        for s in range(S):
            rs_send_r[s, 0, :, :] = t_lo(my, s)
            rs_send_l[s, 0, :, :] = t_hi(lax.rem(my + 2, N_DEV), s)
            rs[s, 0] = start_rs(s, 0)

        w_bf = w_ref[:, :].astype(bf16)

        for h in range(H - 1):
            for s in range(S):
                r, l = rs[s, h]
                r.wait()
                l.wait()
                r_idx = lax.rem(my - h - 1 + 2 * N_DEV, N_DEV)
                l_idx = lax.rem(my + h + 3, N_DEV)
                rs_send_r[s, h + 1, :, :] = rs_recv_r[s, h] + t_lo(r_idx, s)
                rs_send_l[s, h + 1, :, :] = rs_recv_l[s, h] + t_hi(l_idx, s)
                rs[s, h + 1] = start_rs(s, h + 1)

        my_out = lax.rem(my + 1, N_DEV)
        for s in range(S):
            r, l = rs[s, H - 1]
            r.wait()
            l.wait()
            r_idx = lax.rem(my - H + 2 * N_DEV, N_DEV)
            l_idx = lax.rem(my + H + 2, N_DEV)
            sum_lo = rs_recv_r[s, H - 1] + t_lo(r_idx, s)
            sum_hi = rs_recv_l[s, H - 1] + t_hi(l_idx, s)
            out_sub = (
                jnp.dot(sum_lo, w_bf[:kh, :],
                        preferred_element_type=jnp.float32)
                + jnp.dot(sum_hi, w_bf[kh:, :],
                          preferred_element_type=jnp.float32)
            ).astype(bf16)
            out_ref[out_rows(my_out, s), :] = out_sub
            ag[s, 0] = start_ag(s, 0)

        for h in range(H - 1):
            for s in range(S):
                r, l = ag[s, h]
                r.wait()
                l.wait()
                ag[s, h + 1] = start_ag(s, h + 1)
        for s in range(S):
            r, l = ag[s, H - 1]
            r.wait()
            l.wait()

    return pl.pallas_call(
        body,
        out_shape=jax.ShapeDtypeStruct((m_per, n), bf16),
        in_specs=[
            pl.BlockSpec(memory_space=pltpu.VMEM),
            pl.BlockSpec(memory_space=pltpu.VMEM),
        ],
        out_specs=pl.BlockSpec(memory_space=pltpu.VMEM),
        scratch_shapes=[
            pltpu.VMEM((S, H, sh, kh), bf16),
            pltpu.VMEM((S, H, sh, kh), bf16),
            pltpu.VMEM((S, H, sh, kh), bf16),
            pltpu.VMEM((S, H, sh, kh), bf16),
            pltpu.SemaphoreType.DMA((S, H)),
            pltpu.SemaphoreType.DMA((S, H)),
            pltpu.SemaphoreType.DMA((S, H)),
            pltpu.SemaphoreType.DMA((S, H)),
            pltpu.SemaphoreType.DMA((S, H)),
            pltpu.SemaphoreType.DMA((S, H)),
            pltpu.SemaphoreType.DMA((S, H)),
            pltpu.SemaphoreType.DMA((S, H)),
        ],
        compiler_params=pltpu.CompilerParams(collective_id=0),
    )(t, W)
